# Initial kernel scaffold; baseline (speedup 1.0000x reference)
import jax
import jax.numpy as jnp
from jax.experimental import pallas as pl


def kernel(x, edge_index, pert_W, pert_b, basal_W, basal_b, emb, embtrans_W, embtrans_b, pbt_W, pbt_b, gcn0_W, gcn0_b, gcn1_W, gcn1_b, rec_W1, rec_b1, rec_W2, rec_b2, rec_W3, rec_b3):
    raise NotImplementedError("write your pallas kernel here")


# speedup vs baseline: 18.3917x; 18.3917x over previous
import functools

import jax
import jax.numpy as jnp
from jax import lax
from jax.experimental import pallas as pl
from jax.experimental.pallas import tpu as pltpu
from jax.experimental.pallas import tpu_sc as plsc

F32 = jnp.float32
_NC = 2
_NS = 16
_SW = 128
_BL = 8


def _round_up(a, b):
    return (a + b - 1) // b * b


def _dot(a, b):
    return jnp.dot(a.astype(jnp.bfloat16), b.astype(jnp.bfloat16),
                   preferred_element_type=F32)


def _mesh():
    return plsc.VectorSubcoreMesh(core_axis_name="c", subcore_axis_name="s")


def _deg_call(n, e_pad, nrows, dst2d, ones_hbm, z1d_hbm):
    rows_total = e_pad // _SW
    half_rows = rows_total // 2
    tile_rows = half_rows // _NS
    nblk = tile_rows // _BL
    stripe = nrows // _NS
    so = _round_up(-(-n // _NS), 16)
    last = n - so * (_NS - 1)

    def body(dst_r, ones_r, z_r, out_r, idx_v, ones_v, zer_v, obuf, deg_sh,
             sem):
        c = lax.axis_index("c")
        s = lax.axis_index("s")
        pltpu.sync_copy(ones_r, ones_v)
        pltpu.sync_copy(z_r, zer_v)
        base = s * stripe
        pltpu.sync_copy(zer_v, deg_sh.at[pl.ds(base, 2048)])
        pltpu.sync_copy(zer_v.at[pl.ds(0, stripe - 2048)],
                        deg_sh.at[pl.ds(base + 2048, stripe - 2048)])
        plsc.subcore_barrier()

        row0 = c * half_rows + s * tile_rows

        def blk(b, carry):
            pltpu.sync_copy(dst_r.at[pl.ds(row0 + b * _BL, _BL)], idx_v)
            cps = [pltpu.async_copy(ones_v, deg_sh.at[idx_v.at[j]], sem,
                                    add=True)
                   for j in range(_BL)]
            for cp in cps:
                cp.wait()
            return carry

        lax.fori_loop(0, nblk, blk, 0)
        plsc.subcore_barrier()

        @pl.when(s < _NS - 1)
        def _():
            pltpu.sync_copy(deg_sh.at[pl.ds(s * so, so)],
                            obuf.at[pl.ds(0, so)])
            pltpu.sync_copy(obuf.at[pl.ds(0, so)],
                            out_r.at[pl.ds(c * n + s * so, so)])

        @pl.when(s == _NS - 1)
        def _():
            pltpu.sync_copy(deg_sh.at[pl.ds((_NS - 1) * so, last)],
                            obuf.at[pl.ds(0, last)])
            pltpu.sync_copy(obuf.at[pl.ds(0, last)],
                            out_r.at[pl.ds(c * n + (_NS - 1) * so, last)])

    fn = pl.kernel(
        body,
        out_type=jax.ShapeDtypeStruct((2 * n,), F32),
        mesh=_mesh(),
        compiler_params=pltpu.CompilerParams(use_tc_tiling_on_sc=False),
        scratch_types=[
            pltpu.VMEM((_BL, _SW), jnp.int32),
            pltpu.VMEM((_SW,), F32),
            pltpu.VMEM((2048,), F32),
            pltpu.VMEM((_round_up(-(-n // _NS), 16),), F32),
            pltpu.VMEM_SHARED((nrows,), F32),
            pltpu.SemaphoreType.DMA,
        ],
    )
    return fn(dst2d, ones_hbm, z1d_hbm)


def _agg_call(n, e_pad, nrows, g_lo, g_hi, src2d, dst2d, z2d_hbm):
    rows_total = e_pad // _SW
    tile_rows = rows_total // _NS
    nblk = tile_rows // _BL
    stripe = nrows // _NS
    zfull = stripe // _SW
    zrem = stripe - zfull * _SW
    on = _round_up(-(-n // _NS), 16)
    on_last = n - on * (_NS - 1)

    def body(glo_r, ghi_r, src_r, dst_r, z_r, olo_r, ohi_r,
             srcb, dstb, rows0, rows1, zbuf, acc_sh, gsem, ssem):
        c = lax.axis_index("c")
        s = lax.axis_index("s")
        pltpu.sync_copy(z_r, zbuf)
        base = s * stripe
        for k in range(zfull):
            pltpu.sync_copy(zbuf, acc_sh.at[pl.ds(base + k * _SW, _SW)])
        if zrem:
            pltpu.sync_copy(zbuf.at[pl.ds(0, zrem)],
                            acc_sh.at[pl.ds(base + zfull * _SW, zrem)])
        plsc.subcore_barrier()

        row0 = s * tile_rows
        rows = [rows0, rows1]

        def run(g_ref):
            def blk(b, carry):
                r = row0 + b * _BL
                pltpu.sync_copy(src_r.at[pl.ds(r, _BL)], srcb)
                pltpu.sync_copy(dst_r.at[pl.ds(r, _BL)], dstb)
                scat = [None, None]
                for j in range(_BL):
                    bi = j % 2
                    if scat[bi] is not None:
                        scat[bi].wait()
                    pltpu.async_copy(g_ref.at[srcb.at[j]], rows[bi],
                                     gsem).wait()
                    scat[bi] = pltpu.async_copy(rows[bi],
                                                acc_sh.at[dstb.at[j]],
                                                ssem, add=True)
                scat[0].wait()
                scat[1].wait()
                return carry
            lax.fori_loop(0, nblk, blk, 0)

        @pl.when(c == 0)
        def _():
            run(glo_r)

        @pl.when(c == 1)
        def _():
            run(ghi_r)

        plsc.subcore_barrier()

        def flush(out_ref):
            rb = s * on

            def piece(nr):
                nf = nr // _SW
                rr = nr - nf * _SW
                for k in range(nf):
                    pltpu.sync_copy(acc_sh.at[pl.ds(rb + k * _SW, _SW)],
                                    rows0)
                    pltpu.sync_copy(rows0,
                                    out_ref.at[pl.ds(rb + k * _SW, _SW)])
                if rr:
                    pltpu.sync_copy(acc_sh.at[pl.ds(rb + nf * _SW, rr)],
                                    rows0.at[pl.ds(0, rr)])
                    pltpu.sync_copy(rows0.at[pl.ds(0, rr)],
                                    out_ref.at[pl.ds(rb + nf * _SW, rr)])

            @pl.when(s < _NS - 1)
            def _():
                piece(on)

            @pl.when(s == _NS - 1)
            def _():
                piece(on_last)

        @pl.when(c == 0)
        def _():
            flush(olo_r)

        @pl.when(c == 1)
        def _():
            flush(ohi_r)

    fn = pl.kernel(
        body,
        out_type=[jax.ShapeDtypeStruct((n, 32), F32),
                  jax.ShapeDtypeStruct((n, 32), F32)],
        mesh=_mesh(),
        compiler_params=pltpu.CompilerParams(use_tc_tiling_on_sc=False),
        scratch_types=[
            pltpu.VMEM((_BL, _SW), jnp.int32),
            pltpu.VMEM((_BL, _SW), jnp.int32),
            pltpu.VMEM((_SW, 32), F32),
            pltpu.VMEM((_SW, 32), F32),
            pltpu.VMEM((_SW, 32), F32),
            pltpu.VMEM_SHARED((nrows, 32), F32),
            pltpu.SemaphoreType.DMA,
            pltpu.SemaphoreType.DMA,
        ],
    )
    return fn(g_lo, g_hi, src2d, dst2d, z2d_hbm)


def _feat_call(n, ng, ngr, hdim, x, d0, d1, emb, etW, etb, pbW, pbb,
               baW, bab, peW, peb):
    def body(x_r, d0_r, d1_r, emb_r, etW_r, etb_r, pbW_r, pbb_r,
             baW_r, bab_r, peW_r, peb_r, glo_r, ghi_r, dinv_r):
        gb = x_r[:, 0:1]
        pert = x_r[:, 1:2]
        pe = pert * peW_r[...] + peb_r[...]
        be = gb * baW_r[...] + bab_r[...]
        be2 = _dot(jnp.concatenate([emb_r[...], be], axis=1),
                   etW_r[...]) + etb_r[...]
        h0 = _dot(jnp.concatenate([be2, pe], axis=1),
                  pbW_r[...]) + pbb_r[...]
        dinv = 1.0 / jnp.sqrt(d0_r[...] + d1_r[...] + 1.0)
        g0 = h0 * dinv
        glo_r[...] = g0[:, 0:32]
        ghi_r[...] = g0[:, 32:64]
        dinv_r[...] = dinv

    full = lambda shp: pl.BlockSpec(shp, lambda g: tuple(0 for _ in shp))
    blk = lambda cdim: pl.BlockSpec((ng, cdim), lambda g: (g, 0))
    return pl.pallas_call(
        body,
        grid=(ngr,),
        in_specs=[blk(2), blk(1), blk(1), full((ng, hdim)),
                  full(etW.shape), full(etb.shape),
                  full(pbW.shape), full(pbb.shape),
                  full(baW.shape), full(bab.shape),
                  full(peW.shape), full(peb.shape)],
        out_specs=[blk(32), blk(32), blk(1)],
        out_shape=[jax.ShapeDtypeStruct((n, 32), F32),
                   jax.ShapeDtypeStruct((n, 32), F32),
                   jax.ShapeDtypeStruct((n, 1), F32)],
    )(x, d0, d1, emb, etW, etb, pbW, pbb, baW, bab, peW, peb)


def _gcn_post0_call(n, hdim, rb, alo, ahi, glo, ghi, dinv, w_m, b_v):
    grid = n // rb

    def body(alo_r, ahi_r, glo_r, ghi_r, dv_r, w_r, b_r, olo_r, ohi_r):
        t = jnp.concatenate([alo_r[...] + glo_r[...],
                             ahi_r[...] + ghi_r[...]], axis=1) * dv_r[...]
        h1 = jnp.maximum(_dot(t, w_r[...]) + b_r[...], 0.0)
        g1 = h1 * dv_r[...]
        olo_r[...] = g1[:, 0:32]
        ohi_r[...] = g1[:, 32:64]

    full = lambda shp: pl.BlockSpec(shp, lambda g: tuple(0 for _ in shp))
    blk = lambda cdim: pl.BlockSpec((rb, cdim), lambda g: (g, 0))
    return pl.pallas_call(
        body,
        grid=(grid,),
        in_specs=[blk(32), blk(32), blk(32), blk(32), blk(1),
                  full(w_m.shape), full(b_v.shape)],
        out_specs=[blk(32), blk(32)],
        out_shape=[jax.ShapeDtypeStruct((n, 32), F32),
                   jax.ShapeDtypeStruct((n, 32), F32)],
    )(alo, ahi, glo, ghi, dinv, w_m, b_v)


def _gcn_post1_call(n, hdim, rb, alo, ahi, glo, ghi, dinv, w_m, b_v):
    grid = n // rb

    def body(alo_r, ahi_r, glo_r, ghi_r, dv_r, w_r, b_r, h2_r, st_r):
        t = jnp.concatenate([alo_r[...] + glo_r[...],
                             ahi_r[...] + ghi_r[...]], axis=1) * dv_r[...]
        h2 = _dot(t, w_r[...]) + b_r[...]
        h2_r[...] = h2

        @pl.when(pl.program_id(0) == 0)
        def _():
            st_r[...] = jnp.zeros_like(st_r)

        st_r[...] += jnp.concatenate(
            [jnp.sum(h2, axis=0, keepdims=True),
             jnp.sum(h2 * h2, axis=0, keepdims=True)], axis=0)

    full = lambda shp: pl.BlockSpec(shp, lambda g: tuple(0 for _ in shp))
    blk = lambda cdim: pl.BlockSpec((rb, cdim), lambda g: (g, 0))
    return pl.pallas_call(
        body,
        grid=(grid,),
        in_specs=[blk(32), blk(32), blk(32), blk(32), blk(1),
                  full(w_m.shape), full(b_v.shape)],
        out_specs=[blk(hdim), full((2, hdim))],
        out_shape=[jax.ShapeDtypeStruct((n, hdim), F32),
                   jax.ShapeDtypeStruct((2, hdim), F32)],
    )(alo, ahi, glo, ghi, dinv, w_m, b_v)


def _bn_mm_call(n, rb, cin, cout, h, st, w_m, b_v):
    grid = n // rb
    inv_n = 1.0 / n

    def body(h_r, st_r, w_r, b_r, z_r, so_r):
        m = st_r[0:1, :] * inv_n
        var = st_r[1:2, :] * inv_n - m * m
        r = 1.0 / jnp.sqrt(var + 1e-5)
        hb = jnp.maximum((h_r[...] - m) * r, 0.0)
        z = _dot(hb, w_r[...]) + b_r[...]
        z_r[...] = z

        @pl.when(pl.program_id(0) == 0)
        def _():
            so_r[...] = jnp.zeros_like(so_r)

        so_r[...] += jnp.concatenate(
            [jnp.sum(z, axis=0, keepdims=True),
             jnp.sum(z * z, axis=0, keepdims=True)], axis=0)

    full = lambda shp: pl.BlockSpec(shp, lambda g: tuple(0 for _ in shp))
    return pl.pallas_call(
        body,
        grid=(grid,),
        in_specs=[pl.BlockSpec((rb, cin), lambda g: (g, 0)),
                  full((2, cin)), full(w_m.shape), full(b_v.shape)],
        out_specs=[pl.BlockSpec((rb, cout), lambda g: (g, 0)),
                   full((2, cout))],
        out_shape=[jax.ShapeDtypeStruct((n, cout), F32),
                   jax.ShapeDtypeStruct((2, cout), F32)],
    )(h, st, w_m, b_v)


def _tail_call(n, rb, cin, z2, st, w_m, b_v, gb):
    inv_n = 1.0 / n
    grid = n // rb

    def body(z_r, st_r, w_r, b_r, gb_r, o_r):
        m = st_r[0:1, :] * inv_n
        var = st_r[1:2, :] * inv_n - m * m
        r = 1.0 / jnp.sqrt(var + 1e-5)
        zb = jnp.maximum((z_r[...] - m) * r, 0.0)
        o_r[...] = _dot(zb, w_r[...]) + b_r[...] + gb_r[...]

    full = lambda shp: pl.BlockSpec(shp, lambda g: tuple(0 for _ in shp))
    return pl.pallas_call(
        body,
        grid=(grid,),
        in_specs=[pl.BlockSpec((rb, cin), lambda g: (g, 0)),
                  full((2, cin)), full(w_m.shape), full(b_v.shape),
                  pl.BlockSpec((rb, 1), lambda g: (g, 0))],
        out_specs=pl.BlockSpec((rb, 1), lambda g: (g, 0)),
        out_shape=jax.ShapeDtypeStruct((n, 1), F32),
    )(z2, st, w_m, b_v, gb)


def kernel(x, edge_index, pert_W, pert_b, basal_W, basal_b, emb,
           embtrans_W, embtrans_b, pbt_W, pbt_b, gcn0_W, gcn0_b,
           gcn1_W, gcn1_b, rec_W1, rec_b1, rec_W2, rec_b2, rec_W3, rec_b3):
    n = x.shape[0]
    e = edge_index.shape[1]
    ng = emb.shape[0]
    ngr = n // ng
    hdim = emb.shape[1]

    half_pad = _round_up(e - e // 2, _NS * _SW * _BL)
    e_pad = 2 * half_pad
    nrows = _round_up(n + 64, _NS * _SW)

    npad = e_pad - e
    pi = jnp.arange(npad, dtype=jnp.int32)
    src_p = jnp.concatenate([edge_index[0], pi % 32])
    dst_p = jnp.concatenate([edge_index[1], n + (pi % 32)])
    src2d = src_p.reshape(-1, _SW)
    dst2d = dst_p.reshape(-1, _SW)

    ones128 = jnp.ones((_SW,), F32)
    z1d = jnp.zeros((2048,), F32)
    z2d = jnp.zeros((_SW, 32), F32)

    degp = _deg_call(n, e_pad, nrows, dst2d, ones128, z1d)
    d0 = degp[0:n].reshape(n, 1)
    d1 = degp[n:2 * n].reshape(n, 1)

    glo, ghi, dinv = _feat_call(
        n, ng, ngr, hdim, x, d0, d1, emb, embtrans_W,
        embtrans_b.reshape(1, -1), pbt_W, pbt_b.reshape(1, -1),
        basal_W, basal_b.reshape(1, -1), pert_W, pert_b.reshape(1, -1))

    def _agg(gl, gh):
        return _agg_call(n, e_pad, nrows, gl, gh, src2d, dst2d, z2d)

    rb = n // 25
    alo0, ahi0 = _agg(glo, ghi)
    g1lo, g1hi = _gcn_post0_call(n, hdim, rb, alo0, ahi0, glo, ghi, dinv,
                                 gcn0_W, gcn0_b.reshape(1, -1))
    alo1, ahi1 = _agg(g1lo, g1hi)
    h2, st1 = _gcn_post1_call(n, hdim, rb, alo1, ahi1, g1lo, g1hi, dinv,
                              gcn1_W, gcn1_b.reshape(1, -1))
    z1, st2 = _bn_mm_call(n, rb, hdim, 2 * hdim, h2, st1,
                          rec_W1, rec_b1.reshape(1, -1))
    z2, st3 = _bn_mm_call(n, rb, 2 * hdim, hdim, z1, st2,
                          rec_W2, rec_b2.reshape(1, -1))
    o = _tail_call(n, rb, hdim, z2, st3, rec_W3, rec_b3.reshape(1, -1),
                   x[:, 0:1])
    return o.reshape(ngr, ng)


# speedup vs baseline: 25.8186x; 1.4038x over previous
import functools

import jax
import jax.numpy as jnp
from jax import lax
from jax.experimental import pallas as pl
from jax.experimental.pallas import tpu as pltpu
from jax.experimental.pallas import tpu_sc as plsc

F32 = jnp.float32
_NC = 2
_NS = 16
_SW = 128
_BL = 8


def _round_up(a, b):
    return (a + b - 1) // b * b


def _dot(a, b):
    return jnp.dot(a.astype(jnp.bfloat16), b.astype(jnp.bfloat16),
                   preferred_element_type=F32)


def _mesh():
    return plsc.VectorSubcoreMesh(core_axis_name="c", subcore_axis_name="s")


def _deg_call(n, e_pad, nrows, dst2d, ones_hbm, z1d_hbm):
    rows_total = e_pad // _SW
    half_rows = rows_total // 2
    tile_rows = half_rows // _NS
    nblk = tile_rows // _BL
    stripe = nrows // _NS
    so = _round_up(-(-n // _NS), 16)
    last = n - so * (_NS - 1)

    def body(dst_r, ones_r, z_r, out_r, idx_v, ones_v, zer_v, obuf, deg_sh,
             sem):
        c = lax.axis_index("c")
        s = lax.axis_index("s")
        pltpu.sync_copy(ones_r, ones_v)
        pltpu.sync_copy(z_r, zer_v)
        base = s * stripe
        pltpu.sync_copy(zer_v, deg_sh.at[pl.ds(base, 2048)])
        pltpu.sync_copy(zer_v.at[pl.ds(0, stripe - 2048)],
                        deg_sh.at[pl.ds(base + 2048, stripe - 2048)])
        plsc.subcore_barrier()

        row0 = c * half_rows + s * tile_rows

        def blk(b, carry):
            pltpu.sync_copy(dst_r.at[pl.ds(row0 + b * _BL, _BL)], idx_v)
            cps = [pltpu.async_copy(ones_v, deg_sh.at[idx_v.at[j]], sem,
                                    add=True)
                   for j in range(_BL)]
            for cp in cps:
                cp.wait()
            return carry

        lax.fori_loop(0, nblk, blk, 0)
        plsc.subcore_barrier()

        @pl.when(s < _NS - 1)
        def _():
            pltpu.sync_copy(deg_sh.at[pl.ds(s * so, so)],
                            obuf.at[pl.ds(0, so)])
            pltpu.sync_copy(obuf.at[pl.ds(0, so)],
                            out_r.at[pl.ds(c * n + s * so, so)])

        @pl.when(s == _NS - 1)
        def _():
            pltpu.sync_copy(deg_sh.at[pl.ds((_NS - 1) * so, last)],
                            obuf.at[pl.ds(0, last)])
            pltpu.sync_copy(obuf.at[pl.ds(0, last)],
                            out_r.at[pl.ds(c * n + (_NS - 1) * so, last)])

    fn = pl.kernel(
        body,
        out_type=jax.ShapeDtypeStruct((2 * n,), F32),
        mesh=_mesh(),
        compiler_params=pltpu.CompilerParams(use_tc_tiling_on_sc=False),
        scratch_types=[
            pltpu.VMEM((_BL, _SW), jnp.int32),
            pltpu.VMEM((_SW,), F32),
            pltpu.VMEM((2048,), F32),
            pltpu.VMEM((_round_up(-(-n // _NS), 16),), F32),
            pltpu.VMEM_SHARED((nrows,), F32),
            pltpu.SemaphoreType.DMA,
        ],
    )
    return fn(dst2d, ones_hbm, z1d_hbm)


_BLA = 16
_NBUF = 4
_GDEPTH = 3


def _agg_call(n, e_pad, nrows, g_lo, g_hi, src2d, dst2d, z2d_hbm):
    rows_total = e_pad // _SW
    tile_rows = rows_total // _NS
    nblk = tile_rows // _BLA
    stripe = nrows // _NS
    zfull = stripe // _SW
    zrem = stripe - zfull * _SW
    on = _round_up(-(-n // _NS), 16)
    on_last = n - on * (_NS - 1)

    def body(glo_r, ghi_r, src_r, dst_r, z_r, olo_r, ohi_r,
             srcb, dstb, rows, zbuf, acc_sh, gsems, ssems):
        c = lax.axis_index("c")
        s = lax.axis_index("s")
        pltpu.sync_copy(z_r, zbuf)
        base = s * stripe
        for k in range(zfull):
            pltpu.sync_copy(zbuf, acc_sh.at[pl.ds(base + k * _SW, _SW)])
        if zrem:
            pltpu.sync_copy(zbuf.at[pl.ds(0, zrem)],
                            acc_sh.at[pl.ds(base + zfull * _SW, zrem)])
        plsc.subcore_barrier()

        row0 = s * tile_rows

        def run(g_ref):
            def blk(b, carry):
                r = row0 + b * _BLA
                pltpu.sync_copy(src_r.at[pl.ds(r, _BLA)], srcb)
                pltpu.sync_copy(dst_r.at[pl.ds(r, _BLA)], dstb)
                gat = [None] * _NBUF
                scat = [None] * _NBUF

                def scatter(j):
                    pb = j % _NBUF
                    gat[pb].wait()
                    scat[pb] = pltpu.async_copy(
                        rows.at[pb], acc_sh.at[dstb.at[j]],
                        ssems.at[pb], add=True)

                for j in range(_BLA):
                    bi = j % _NBUF
                    if scat[bi] is not None:
                        scat[bi].wait()
                    gat[bi] = pltpu.async_copy(
                        g_ref.at[srcb.at[j]], rows.at[bi], gsems.at[bi])
                    if j >= _GDEPTH - 1:
                        scatter(j - (_GDEPTH - 1))
                for j in range(_BLA - (_GDEPTH - 1), _BLA):
                    scatter(j)
                for bi in range(_NBUF):
                    if scat[bi] is not None:
                        scat[bi].wait()
                return carry
            lax.fori_loop(0, nblk, blk, 0)

        @pl.when(c == 0)
        def _():
            run(glo_r)

        @pl.when(c == 1)
        def _():
            run(ghi_r)

        plsc.subcore_barrier()

        def flush(out_ref):
            rb = s * on

            def piece(nr):
                nf = nr // _SW
                rr = nr - nf * _SW
                for k in range(nf):
                    pltpu.sync_copy(acc_sh.at[pl.ds(rb + k * _SW, _SW)],
                                    zbuf)
                    pltpu.sync_copy(zbuf,
                                    out_ref.at[pl.ds(rb + k * _SW, _SW)])
                if rr:
                    pltpu.sync_copy(acc_sh.at[pl.ds(rb + nf * _SW, rr)],
                                    zbuf.at[pl.ds(0, rr)])
                    pltpu.sync_copy(zbuf.at[pl.ds(0, rr)],
                                    out_ref.at[pl.ds(rb + nf * _SW, rr)])

            @pl.when(s < _NS - 1)
            def _():
                piece(on)

            @pl.when(s == _NS - 1)
            def _():
                piece(on_last)

        @pl.when(c == 0)
        def _():
            flush(olo_r)

        @pl.when(c == 1)
        def _():
            flush(ohi_r)

    fn = pl.kernel(
        body,
        out_type=[jax.ShapeDtypeStruct((n, 32), F32),
                  jax.ShapeDtypeStruct((n, 32), F32)],
        mesh=_mesh(),
        compiler_params=pltpu.CompilerParams(use_tc_tiling_on_sc=False),
        scratch_types=[
            pltpu.VMEM((_BLA, _SW), jnp.int32),
            pltpu.VMEM((_BLA, _SW), jnp.int32),
            pltpu.VMEM((_NBUF, _SW, 32), F32),
            pltpu.VMEM((_SW, 32), F32),
            pltpu.VMEM_SHARED((nrows, 32), F32),
            pltpu.SemaphoreType.DMA((_NBUF,)),
            pltpu.SemaphoreType.DMA((_NBUF,)),
        ],
    )
    return fn(g_lo, g_hi, src2d, dst2d, z2d_hbm)


def _feat_call(n, ng, ngr, hdim, x, d0, d1, emb, etW, etb, pbW, pbb,
               baW, bab, peW, peb):
    def body(x_r, d0_r, d1_r, emb_r, etW_r, etb_r, pbW_r, pbb_r,
             baW_r, bab_r, peW_r, peb_r, glo_r, ghi_r, dinv_r):
        gb = x_r[:, 0:1]
        pert = x_r[:, 1:2]
        pe = pert * peW_r[...] + peb_r[...]
        be = gb * baW_r[...] + bab_r[...]
        be2 = _dot(jnp.concatenate([emb_r[...], be], axis=1),
                   etW_r[...]) + etb_r[...]
        h0 = _dot(jnp.concatenate([be2, pe], axis=1),
                  pbW_r[...]) + pbb_r[...]
        dinv = 1.0 / jnp.sqrt(d0_r[...] + d1_r[...] + 1.0)
        g0 = h0 * dinv
        glo_r[...] = g0[:, 0:32]
        ghi_r[...] = g0[:, 32:64]
        dinv_r[...] = dinv

    full = lambda shp: pl.BlockSpec(shp, lambda g: tuple(0 for _ in shp))
    blk = lambda cdim: pl.BlockSpec((ng, cdim), lambda g: (g, 0))
    return pl.pallas_call(
        body,
        grid=(ngr,),
        in_specs=[blk(2), blk(1), blk(1), full((ng, hdim)),
                  full(etW.shape), full(etb.shape),
                  full(pbW.shape), full(pbb.shape),
                  full(baW.shape), full(bab.shape),
                  full(peW.shape), full(peb.shape)],
        out_specs=[blk(32), blk(32), blk(1)],
        out_shape=[jax.ShapeDtypeStruct((n, 32), F32),
                   jax.ShapeDtypeStruct((n, 32), F32),
                   jax.ShapeDtypeStruct((n, 1), F32)],
    )(x, d0, d1, emb, etW, etb, pbW, pbb, baW, bab, peW, peb)


def _gcn_post0_call(n, hdim, rb, alo, ahi, glo, ghi, dinv, w_m, b_v):
    grid = n // rb

    def body(alo_r, ahi_r, glo_r, ghi_r, dv_r, w_r, b_r, olo_r, ohi_r):
        t = jnp.concatenate([alo_r[...] + glo_r[...],
                             ahi_r[...] + ghi_r[...]], axis=1) * dv_r[...]
        h1 = jnp.maximum(_dot(t, w_r[...]) + b_r[...], 0.0)
        g1 = h1 * dv_r[...]
        olo_r[...] = g1[:, 0:32]
        ohi_r[...] = g1[:, 32:64]

    full = lambda shp: pl.BlockSpec(shp, lambda g: tuple(0 for _ in shp))
    blk = lambda cdim: pl.BlockSpec((rb, cdim), lambda g: (g, 0))
    return pl.pallas_call(
        body,
        grid=(grid,),
        in_specs=[blk(32), blk(32), blk(32), blk(32), blk(1),
                  full(w_m.shape), full(b_v.shape)],
        out_specs=[blk(32), blk(32)],
        out_shape=[jax.ShapeDtypeStruct((n, 32), F32),
                   jax.ShapeDtypeStruct((n, 32), F32)],
    )(alo, ahi, glo, ghi, dinv, w_m, b_v)


def _gcn_post1_call(n, hdim, rb, alo, ahi, glo, ghi, dinv, w_m, b_v):
    grid = n // rb

    def body(alo_r, ahi_r, glo_r, ghi_r, dv_r, w_r, b_r, h2_r, st_r):
        t = jnp.concatenate([alo_r[...] + glo_r[...],
                             ahi_r[...] + ghi_r[...]], axis=1) * dv_r[...]
        h2 = _dot(t, w_r[...]) + b_r[...]
        h2_r[...] = h2

        @pl.when(pl.program_id(0) == 0)
        def _():
            st_r[...] = jnp.zeros_like(st_r)

        st_r[...] += jnp.concatenate(
            [jnp.sum(h2, axis=0, keepdims=True),
             jnp.sum(h2 * h2, axis=0, keepdims=True)], axis=0)

    full = lambda shp: pl.BlockSpec(shp, lambda g: tuple(0 for _ in shp))
    blk = lambda cdim: pl.BlockSpec((rb, cdim), lambda g: (g, 0))
    return pl.pallas_call(
        body,
        grid=(grid,),
        in_specs=[blk(32), blk(32), blk(32), blk(32), blk(1),
                  full(w_m.shape), full(b_v.shape)],
        out_specs=[blk(hdim), full((2, hdim))],
        out_shape=[jax.ShapeDtypeStruct((n, hdim), F32),
                   jax.ShapeDtypeStruct((2, hdim), F32)],
    )(alo, ahi, glo, ghi, dinv, w_m, b_v)


def _bn_mm_call(n, rb, cin, cout, h, st, w_m, b_v):
    grid = n // rb
    inv_n = 1.0 / n

    def body(h_r, st_r, w_r, b_r, z_r, so_r):
        m = st_r[0:1, :] * inv_n
        var = st_r[1:2, :] * inv_n - m * m
        r = 1.0 / jnp.sqrt(var + 1e-5)
        hb = jnp.maximum((h_r[...] - m) * r, 0.0)
        z = _dot(hb, w_r[...]) + b_r[...]
        z_r[...] = z

        @pl.when(pl.program_id(0) == 0)
        def _():
            so_r[...] = jnp.zeros_like(so_r)

        so_r[...] += jnp.concatenate(
            [jnp.sum(z, axis=0, keepdims=True),
             jnp.sum(z * z, axis=0, keepdims=True)], axis=0)

    full = lambda shp: pl.BlockSpec(shp, lambda g: tuple(0 for _ in shp))
    return pl.pallas_call(
        body,
        grid=(grid,),
        in_specs=[pl.BlockSpec((rb, cin), lambda g: (g, 0)),
                  full((2, cin)), full(w_m.shape), full(b_v.shape)],
        out_specs=[pl.BlockSpec((rb, cout), lambda g: (g, 0)),
                   full((2, cout))],
        out_shape=[jax.ShapeDtypeStruct((n, cout), F32),
                   jax.ShapeDtypeStruct((2, cout), F32)],
    )(h, st, w_m, b_v)


def _tail_call(n, rb, cin, z2, st, w_m, b_v, gb):
    inv_n = 1.0 / n
    grid = n // rb

    def body(z_r, st_r, w_r, b_r, gb_r, o_r):
        m = st_r[0:1, :] * inv_n
        var = st_r[1:2, :] * inv_n - m * m
        r = 1.0 / jnp.sqrt(var + 1e-5)
        zb = jnp.maximum((z_r[...] - m) * r, 0.0)
        o_r[...] = _dot(zb, w_r[...]) + b_r[...] + gb_r[...]

    full = lambda shp: pl.BlockSpec(shp, lambda g: tuple(0 for _ in shp))
    return pl.pallas_call(
        body,
        grid=(grid,),
        in_specs=[pl.BlockSpec((rb, cin), lambda g: (g, 0)),
                  full((2, cin)), full(w_m.shape), full(b_v.shape),
                  pl.BlockSpec((rb, 1), lambda g: (g, 0))],
        out_specs=pl.BlockSpec((rb, 1), lambda g: (g, 0)),
        out_shape=jax.ShapeDtypeStruct((n, 1), F32),
    )(z2, st, w_m, b_v, gb)


def kernel(x, edge_index, pert_W, pert_b, basal_W, basal_b, emb,
           embtrans_W, embtrans_b, pbt_W, pbt_b, gcn0_W, gcn0_b,
           gcn1_W, gcn1_b, rec_W1, rec_b1, rec_W2, rec_b2, rec_W3, rec_b3):
    n = x.shape[0]
    e = edge_index.shape[1]
    ng = emb.shape[0]
    ngr = n // ng
    hdim = emb.shape[1]

    half_pad = _round_up(e - e // 2, _NS * _SW * _BL)
    e_pad = 2 * half_pad
    nrows = _round_up(n + 64, _NS * _SW)

    npad = e_pad - e
    pi = jnp.arange(npad, dtype=jnp.int32)
    src_p = jnp.concatenate([edge_index[0], pi % 32])
    dst_p = jnp.concatenate([edge_index[1], n + (pi % 32)])
    src2d = src_p.reshape(-1, _SW)
    dst2d = dst_p.reshape(-1, _SW)

    ones128 = jnp.ones((_SW,), F32)
    z1d = jnp.zeros((2048,), F32)
    z2d = jnp.zeros((_SW, 32), F32)

    degp = _deg_call(n, e_pad, nrows, dst2d, ones128, z1d)
    d0 = degp[0:n].reshape(n, 1)
    d1 = degp[n:2 * n].reshape(n, 1)

    glo, ghi, dinv = _feat_call(
        n, ng, ngr, hdim, x, d0, d1, emb, embtrans_W,
        embtrans_b.reshape(1, -1), pbt_W, pbt_b.reshape(1, -1),
        basal_W, basal_b.reshape(1, -1), pert_W, pert_b.reshape(1, -1))

    def _agg(gl, gh):
        return _agg_call(n, e_pad, nrows, gl, gh, src2d, dst2d, z2d)

    rb = n // 25
    alo0, ahi0 = _agg(glo, ghi)
    g1lo, g1hi = _gcn_post0_call(n, hdim, rb, alo0, ahi0, glo, ghi, dinv,
                                 gcn0_W, gcn0_b.reshape(1, -1))
    alo1, ahi1 = _agg(g1lo, g1hi)
    h2, st1 = _gcn_post1_call(n, hdim, rb, alo1, ahi1, g1lo, g1hi, dinv,
                              gcn1_W, gcn1_b.reshape(1, -1))
    z1, st2 = _bn_mm_call(n, rb, hdim, 2 * hdim, h2, st1,
                          rec_W1, rec_b1.reshape(1, -1))
    z2, st3 = _bn_mm_call(n, rb, 2 * hdim, hdim, z1, st2,
                          rec_W2, rec_b2.reshape(1, -1))
    o = _tail_call(n, rb, hdim, z2, st3, rec_W3, rec_b3.reshape(1, -1),
                   x[:, 0:1])
    return o.reshape(ngr, ng)


# speedup vs baseline: 26.3741x; 1.0215x over previous
import functools

import jax
import jax.numpy as jnp
from jax import lax
from jax.experimental import pallas as pl
from jax.experimental.pallas import tpu as pltpu
from jax.experimental.pallas import tpu_sc as plsc

F32 = jnp.float32
_NC = 2
_NS = 16
_SW = 128
_BL = 8


def _round_up(a, b):
    return (a + b - 1) // b * b


def _dot(a, b):
    return jnp.dot(a.astype(jnp.bfloat16), b.astype(jnp.bfloat16),
                   preferred_element_type=F32)


def _mesh():
    return plsc.VectorSubcoreMesh(core_axis_name="c", subcore_axis_name="s")


def _deg_call(n, e_pad, nrows, dst2d, ones_hbm, z1d_hbm):
    rows_total = e_pad // _SW
    half_rows = rows_total // 2
    tile_rows = half_rows // _NS
    nblk = tile_rows // _BL
    stripe = nrows // _NS
    so = _round_up(-(-n // _NS), 16)
    last = n - so * (_NS - 1)

    def body(dst_r, ones_r, z_r, out_r, idx_v, ones_v, zer_v, obuf, deg_sh,
             sem):
        c = lax.axis_index("c")
        s = lax.axis_index("s")
        pltpu.sync_copy(ones_r, ones_v)
        pltpu.sync_copy(z_r, zer_v)
        base = s * stripe
        pltpu.sync_copy(zer_v, deg_sh.at[pl.ds(base, 2048)])
        pltpu.sync_copy(zer_v.at[pl.ds(0, stripe - 2048)],
                        deg_sh.at[pl.ds(base + 2048, stripe - 2048)])
        plsc.subcore_barrier()

        row0 = c * half_rows + s * tile_rows

        def blk(b, carry):
            pltpu.sync_copy(dst_r.at[pl.ds(row0 + b * _BL, _BL)], idx_v)
            cps = [pltpu.async_copy(ones_v, deg_sh.at[idx_v.at[j]], sem,
                                    add=True)
                   for j in range(_BL)]
            for cp in cps:
                cp.wait()
            return carry

        lax.fori_loop(0, nblk, blk, 0)
        plsc.subcore_barrier()

        @pl.when(s < _NS - 1)
        def _():
            pltpu.sync_copy(deg_sh.at[pl.ds(s * so, so)],
                            obuf.at[pl.ds(0, so)])
            pltpu.sync_copy(obuf.at[pl.ds(0, so)],
                            out_r.at[pl.ds(c * n + s * so, so)])

        @pl.when(s == _NS - 1)
        def _():
            pltpu.sync_copy(deg_sh.at[pl.ds((_NS - 1) * so, last)],
                            obuf.at[pl.ds(0, last)])
            pltpu.sync_copy(obuf.at[pl.ds(0, last)],
                            out_r.at[pl.ds(c * n + (_NS - 1) * so, last)])

    fn = pl.kernel(
        body,
        out_type=jax.ShapeDtypeStruct((2 * n,), F32),
        mesh=_mesh(),
        compiler_params=pltpu.CompilerParams(use_tc_tiling_on_sc=False),
        scratch_types=[
            pltpu.VMEM((_BL, _SW), jnp.int32),
            pltpu.VMEM((_SW,), F32),
            pltpu.VMEM((2048,), F32),
            pltpu.VMEM((_round_up(-(-n // _NS), 16),), F32),
            pltpu.VMEM_SHARED((nrows,), F32),
            pltpu.SemaphoreType.DMA,
        ],
    )
    return fn(dst2d, ones_hbm, z1d_hbm)


_BLA = 16
_NBUF = 5
_GDEPTH = 4


def _agg_call(n, e_pad, nrows, g_lo, g_hi, src2d, dst2d, z2d_hbm):
    rows_total = e_pad // _SW
    tile_rows = rows_total // _NS
    nblk = tile_rows // _BLA
    stripe = nrows // _NS
    zfull = stripe // _SW
    zrem = stripe - zfull * _SW
    on = _round_up(-(-n // _NS), 16)
    on_last = n - on * (_NS - 1)

    def body(glo_r, ghi_r, src_r, dst_r, z_r, olo_r, ohi_r,
             srcb, dstb, rows, zbuf, acc_sh, gsems, ssems):
        c = lax.axis_index("c")
        s = lax.axis_index("s")
        pltpu.sync_copy(z_r, zbuf)
        base = s * stripe
        for k in range(zfull):
            pltpu.sync_copy(zbuf, acc_sh.at[pl.ds(base + k * _SW, _SW)])
        if zrem:
            pltpu.sync_copy(zbuf.at[pl.ds(0, zrem)],
                            acc_sh.at[pl.ds(base + zfull * _SW, zrem)])
        plsc.subcore_barrier()

        row0 = s * tile_rows

        def run(g_ref):
            def blk(b, carry):
                r = row0 + b * _BLA
                pltpu.sync_copy(src_r.at[pl.ds(r, _BLA)], srcb)
                pltpu.sync_copy(dst_r.at[pl.ds(r, _BLA)], dstb)
                gat = [None] * _NBUF
                scat = [None] * _NBUF

                def scatter(j):
                    pb = j % _NBUF
                    gat[pb].wait()
                    scat[pb] = pltpu.async_copy(
                        rows.at[pb], acc_sh.at[dstb.at[j]],
                        ssems.at[pb], add=True)

                for j in range(_BLA):
                    bi = j % _NBUF
                    if scat[bi] is not None:
                        scat[bi].wait()
                    gat[bi] = pltpu.async_copy(
                        g_ref.at[srcb.at[j]], rows.at[bi], gsems.at[bi])
                    if j >= _GDEPTH - 1:
                        scatter(j - (_GDEPTH - 1))
                for j in range(_BLA - (_GDEPTH - 1), _BLA):
                    scatter(j)
                for bi in range(_NBUF):
                    if scat[bi] is not None:
                        scat[bi].wait()
                return carry
            lax.fori_loop(0, nblk, blk, 0)

        @pl.when(c == 0)
        def _():
            run(glo_r)

        @pl.when(c == 1)
        def _():
            run(ghi_r)

        plsc.subcore_barrier()

        def flush(out_ref):
            rb = s * on

            def piece(nr):
                nf = nr // _SW
                rr = nr - nf * _SW
                for k in range(nf):
                    pltpu.sync_copy(acc_sh.at[pl.ds(rb + k * _SW, _SW)],
                                    zbuf)
                    pltpu.sync_copy(zbuf,
                                    out_ref.at[pl.ds(rb + k * _SW, _SW)])
                if rr:
                    pltpu.sync_copy(acc_sh.at[pl.ds(rb + nf * _SW, rr)],
                                    zbuf.at[pl.ds(0, rr)])
                    pltpu.sync_copy(zbuf.at[pl.ds(0, rr)],
                                    out_ref.at[pl.ds(rb + nf * _SW, rr)])

            @pl.when(s < _NS - 1)
            def _():
                piece(on)

            @pl.when(s == _NS - 1)
            def _():
                piece(on_last)

        @pl.when(c == 0)
        def _():
            flush(olo_r)

        @pl.when(c == 1)
        def _():
            flush(ohi_r)

    fn = pl.kernel(
        body,
        out_type=[jax.ShapeDtypeStruct((n, 32), F32),
                  jax.ShapeDtypeStruct((n, 32), F32)],
        mesh=_mesh(),
        compiler_params=pltpu.CompilerParams(use_tc_tiling_on_sc=False),
        scratch_types=[
            pltpu.VMEM((_BLA, _SW), jnp.int32),
            pltpu.VMEM((_BLA, _SW), jnp.int32),
            pltpu.VMEM((_NBUF, _SW, 32), F32),
            pltpu.VMEM((_SW, 32), F32),
            pltpu.VMEM_SHARED((nrows, 32), F32),
            pltpu.SemaphoreType.DMA((_NBUF,)),
            pltpu.SemaphoreType.DMA((_NBUF,)),
        ],
    )
    return fn(g_lo, g_hi, src2d, dst2d, z2d_hbm)


def _feat_call(n, ng, ngr, hdim, x, d0, d1, emb, etW, etb, pbW, pbb,
               baW, bab, peW, peb):
    def body(x_r, d0_r, d1_r, emb_r, etW_r, etb_r, pbW_r, pbb_r,
             baW_r, bab_r, peW_r, peb_r, glo_r, ghi_r, dinv_r):
        gb = x_r[:, 0:1]
        pert = x_r[:, 1:2]
        pe = pert * peW_r[...] + peb_r[...]
        be = gb * baW_r[...] + bab_r[...]
        be2 = _dot(jnp.concatenate([emb_r[...], be], axis=1),
                   etW_r[...]) + etb_r[...]
        h0 = _dot(jnp.concatenate([be2, pe], axis=1),
                  pbW_r[...]) + pbb_r[...]
        dinv = 1.0 / jnp.sqrt(d0_r[...] + d1_r[...] + 1.0)
        g0 = h0 * dinv
        glo_r[...] = g0[:, 0:32]
        ghi_r[...] = g0[:, 32:64]
        dinv_r[...] = dinv

    full = lambda shp: pl.BlockSpec(shp, lambda g: tuple(0 for _ in shp))
    blk = lambda cdim: pl.BlockSpec((ng, cdim), lambda g: (g, 0))
    return pl.pallas_call(
        body,
        grid=(ngr,),
        in_specs=[blk(2), blk(1), blk(1), full((ng, hdim)),
                  full(etW.shape), full(etb.shape),
                  full(pbW.shape), full(pbb.shape),
                  full(baW.shape), full(bab.shape),
                  full(peW.shape), full(peb.shape)],
        out_specs=[blk(32), blk(32), blk(1)],
        out_shape=[jax.ShapeDtypeStruct((n, 32), F32),
                   jax.ShapeDtypeStruct((n, 32), F32),
                   jax.ShapeDtypeStruct((n, 1), F32)],
    )(x, d0, d1, emb, etW, etb, pbW, pbb, baW, bab, peW, peb)


def _gcn_post0_call(n, hdim, rb, alo, ahi, glo, ghi, dinv, w_m, b_v):
    grid = n // rb

    def body(alo_r, ahi_r, glo_r, ghi_r, dv_r, w_r, b_r, olo_r, ohi_r):
        t = jnp.concatenate([alo_r[...] + glo_r[...],
                             ahi_r[...] + ghi_r[...]], axis=1) * dv_r[...]
        h1 = jnp.maximum(_dot(t, w_r[...]) + b_r[...], 0.0)
        g1 = h1 * dv_r[...]
        olo_r[...] = g1[:, 0:32]
        ohi_r[...] = g1[:, 32:64]

    full = lambda shp: pl.BlockSpec(shp, lambda g: tuple(0 for _ in shp))
    blk = lambda cdim: pl.BlockSpec((rb, cdim), lambda g: (g, 0))
    return pl.pallas_call(
        body,
        grid=(grid,),
        in_specs=[blk(32), blk(32), blk(32), blk(32), blk(1),
                  full(w_m.shape), full(b_v.shape)],
        out_specs=[blk(32), blk(32)],
        out_shape=[jax.ShapeDtypeStruct((n, 32), F32),
                   jax.ShapeDtypeStruct((n, 32), F32)],
    )(alo, ahi, glo, ghi, dinv, w_m, b_v)


def _gcn_post1_call(n, hdim, rb, alo, ahi, glo, ghi, dinv, w_m, b_v):
    grid = n // rb

    def body(alo_r, ahi_r, glo_r, ghi_r, dv_r, w_r, b_r, h2_r, st_r):
        t = jnp.concatenate([alo_r[...] + glo_r[...],
                             ahi_r[...] + ghi_r[...]], axis=1) * dv_r[...]
        h2 = _dot(t, w_r[...]) + b_r[...]
        h2_r[...] = h2

        @pl.when(pl.program_id(0) == 0)
        def _():
            st_r[...] = jnp.zeros_like(st_r)

        st_r[...] += jnp.concatenate(
            [jnp.sum(h2, axis=0, keepdims=True),
             jnp.sum(h2 * h2, axis=0, keepdims=True)], axis=0)

    full = lambda shp: pl.BlockSpec(shp, lambda g: tuple(0 for _ in shp))
    blk = lambda cdim: pl.BlockSpec((rb, cdim), lambda g: (g, 0))
    return pl.pallas_call(
        body,
        grid=(grid,),
        in_specs=[blk(32), blk(32), blk(32), blk(32), blk(1),
                  full(w_m.shape), full(b_v.shape)],
        out_specs=[blk(hdim), full((2, hdim))],
        out_shape=[jax.ShapeDtypeStruct((n, hdim), F32),
                   jax.ShapeDtypeStruct((2, hdim), F32)],
    )(alo, ahi, glo, ghi, dinv, w_m, b_v)


def _bn_mm_call(n, rb, cin, cout, h, st, w_m, b_v):
    grid = n // rb
    inv_n = 1.0 / n

    def body(h_r, st_r, w_r, b_r, z_r, so_r):
        m = st_r[0:1, :] * inv_n
        var = st_r[1:2, :] * inv_n - m * m
        r = 1.0 / jnp.sqrt(var + 1e-5)
        hb = jnp.maximum((h_r[...] - m) * r, 0.0)
        z = _dot(hb, w_r[...]) + b_r[...]
        z_r[...] = z

        @pl.when(pl.program_id(0) == 0)
        def _():
            so_r[...] = jnp.zeros_like(so_r)

        so_r[...] += jnp.concatenate(
            [jnp.sum(z, axis=0, keepdims=True),
             jnp.sum(z * z, axis=0, keepdims=True)], axis=0)

    full = lambda shp: pl.BlockSpec(shp, lambda g: tuple(0 for _ in shp))
    return pl.pallas_call(
        body,
        grid=(grid,),
        in_specs=[pl.BlockSpec((rb, cin), lambda g: (g, 0)),
                  full((2, cin)), full(w_m.shape), full(b_v.shape)],
        out_specs=[pl.BlockSpec((rb, cout), lambda g: (g, 0)),
                   full((2, cout))],
        out_shape=[jax.ShapeDtypeStruct((n, cout), F32),
                   jax.ShapeDtypeStruct((2, cout), F32)],
    )(h, st, w_m, b_v)


def _tail_call(n, rb, cin, z2, st, w_m, b_v, gb):
    inv_n = 1.0 / n
    grid = n // rb

    def body(z_r, st_r, w_r, b_r, gb_r, o_r):
        m = st_r[0:1, :] * inv_n
        var = st_r[1:2, :] * inv_n - m * m
        r = 1.0 / jnp.sqrt(var + 1e-5)
        zb = jnp.maximum((z_r[...] - m) * r, 0.0)
        o_r[...] = _dot(zb, w_r[...]) + b_r[...] + gb_r[...]

    full = lambda shp: pl.BlockSpec(shp, lambda g: tuple(0 for _ in shp))
    return pl.pallas_call(
        body,
        grid=(grid,),
        in_specs=[pl.BlockSpec((rb, cin), lambda g: (g, 0)),
                  full((2, cin)), full(w_m.shape), full(b_v.shape),
                  pl.BlockSpec((rb, 1), lambda g: (g, 0))],
        out_specs=pl.BlockSpec((rb, 1), lambda g: (g, 0)),
        out_shape=jax.ShapeDtypeStruct((n, 1), F32),
    )(z2, st, w_m, b_v, gb)


def kernel(x, edge_index, pert_W, pert_b, basal_W, basal_b, emb,
           embtrans_W, embtrans_b, pbt_W, pbt_b, gcn0_W, gcn0_b,
           gcn1_W, gcn1_b, rec_W1, rec_b1, rec_W2, rec_b2, rec_W3, rec_b3):
    n = x.shape[0]
    e = edge_index.shape[1]
    ng = emb.shape[0]
    ngr = n // ng
    hdim = emb.shape[1]

    half_pad = _round_up(e - e // 2, _NS * _SW * _BL)
    e_pad = 2 * half_pad
    nrows = _round_up(n + 64, _NS * _SW)

    npad = e_pad - e
    pi = jnp.arange(npad, dtype=jnp.int32)
    src_p = jnp.concatenate([edge_index[0], pi % 32])
    dst_p = jnp.concatenate([edge_index[1], n + (pi % 32)])
    src2d = src_p.reshape(-1, _SW)
    dst2d = dst_p.reshape(-1, _SW)

    ones128 = jnp.ones((_SW,), F32)
    z1d = jnp.zeros((2048,), F32)
    z2d = jnp.zeros((_SW, 32), F32)

    degp = _deg_call(n, e_pad, nrows, dst2d, ones128, z1d)
    d0 = degp[0:n].reshape(n, 1)
    d1 = degp[n:2 * n].reshape(n, 1)

    glo, ghi, dinv = _feat_call(
        n, ng, ngr, hdim, x, d0, d1, emb, embtrans_W,
        embtrans_b.reshape(1, -1), pbt_W, pbt_b.reshape(1, -1),
        basal_W, basal_b.reshape(1, -1), pert_W, pert_b.reshape(1, -1))

    def _agg(gl, gh):
        return _agg_call(n, e_pad, nrows, gl, gh, src2d, dst2d, z2d)

    rb = n // 25
    alo0, ahi0 = _agg(glo, ghi)
    g1lo, g1hi = _gcn_post0_call(n, hdim, rb, alo0, ahi0, glo, ghi, dinv,
                                 gcn0_W, gcn0_b.reshape(1, -1))
    alo1, ahi1 = _agg(g1lo, g1hi)
    h2, st1 = _gcn_post1_call(n, hdim, rb, alo1, ahi1, g1lo, g1hi, dinv,
                              gcn1_W, gcn1_b.reshape(1, -1))
    z1, st2 = _bn_mm_call(n, rb, hdim, 2 * hdim, h2, st1,
                          rec_W1, rec_b1.reshape(1, -1))
    z2, st3 = _bn_mm_call(n, rb, 2 * hdim, hdim, z1, st2,
                          rec_W2, rec_b2.reshape(1, -1))
    o = _tail_call(n, rb, hdim, z2, st3, rec_W3, rec_b3.reshape(1, -1),
                   x[:, 0:1])
    return o.reshape(ngr, ng)
